# Initial kernel scaffold; baseline (speedup 1.0000x reference)
import jax
import jax.numpy as jnp
from jax.experimental import pallas as pl


def kernel(x, edge_index, t, W1, b1, Wt, bt, W2, b2):
    raise NotImplementedError("write your pallas kernel here")


# speedup vs baseline: 7.6903x; 7.6903x over previous
import functools

import jax
import jax.numpy as jnp
from jax import lax
from jax.experimental import pallas as pl
from jax.experimental.pallas import tpu as pltpu
from jax.experimental.pallas import tpu_sc as plsc

N = 10000
D = 128
E = 320000
HIDDEN = 64
TEMB = 128

NC = 2
NS = 16
NW = NC * NS
EPW = E // NW
CH = 80
NCHUNK = EPW // CH
NPAD = 10240
RPT = NPAD // NS
BN = 1000
BNC = 1024

_HIGH = lax.Precision.HIGHEST


def _proj_body(x_ref, w1_ref, temb_ref, wt_ref, b1_ref, bt_ref, y1_ref, y2_ref):
    cvec = (
        jnp.dot(temb_ref[...], wt_ref[...], preferred_element_type=jnp.float32,
                precision=_HIGH)
        + b1_ref[...]
        + bt_ref[...]
    )
    x = x_ref[...]
    y1_ref[...] = jnp.dot(x, w1_ref[0:D, :], preferred_element_type=jnp.float32,
                          precision=_HIGH) + cvec
    y2_ref[...] = jnp.dot(x, w1_ref[D:2 * D, :], preferred_element_type=jnp.float32,
                          precision=_HIGH)


_proj = pl.pallas_call(
    _proj_body,
    grid=(N // BN,),
    in_specs=[
        pl.BlockSpec((BN, D), lambda i: (i, 0)),
        pl.BlockSpec((2 * D, HIDDEN), lambda i: (0, 0)),
        pl.BlockSpec((1, TEMB), lambda i: (0, 0)),
        pl.BlockSpec((TEMB, HIDDEN), lambda i: (0, 0)),
        pl.BlockSpec((1, HIDDEN), lambda i: (0, 0)),
        pl.BlockSpec((1, HIDDEN), lambda i: (0, 0)),
    ],
    out_specs=[
        pl.BlockSpec((BN, HIDDEN), lambda i: (i, 0)),
        pl.BlockSpec((BN, HIDDEN), lambda i: (i, 0)),
    ],
    out_shape=[
        jax.ShapeDtypeStruct((N, HIDDEN), jnp.float32),
        jax.ShapeDtypeStruct((N, HIDDEN), jnp.float32),
    ],
)


_sc_mesh = plsc.VectorSubcoreMesh(
    core_axis_name="c", subcore_axis_name="s", num_cores=NC, num_subcores=NS
)


@functools.partial(
    pl.kernel,
    out_type=(
        jax.ShapeDtypeStruct((NC, NPAD, HIDDEN), jnp.float32),
        jax.ShapeDtypeStruct((NC, NPAD, HIDDEN), jnp.float32),
        jax.ShapeDtypeStruct((NC, NPAD, 16), jnp.float32),
        jax.ShapeDtypeStruct((NC, NPAD, 16), jnp.float32),
    ),
    mesh=_sc_mesh,
    compiler_params=pltpu.CompilerParams(use_tc_tiling_on_sc=False),
    scratch_types=(
        pltpu.VMEM_SHARED((NPAD, HIDDEN), jnp.float32),
        pltpu.VMEM_SHARED((NPAD, HIDDEN), jnp.float32),
        pltpu.VMEM_SHARED((NPAD, 16), jnp.float32),
        pltpu.VMEM_SHARED((NPAD, 16), jnp.float32),
        pltpu.VMEM((CH,), jnp.int32),
        pltpu.VMEM((CH,), jnp.int32),
        pltpu.VMEM((CH, HIDDEN), jnp.float32),
        pltpu.VMEM((CH, HIDDEN), jnp.float32),
        pltpu.VMEM((CH, HIDDEN), jnp.float32),
        pltpu.VMEM((CH, 16), jnp.float32),
        pltpu.SemaphoreType.DMA,
        pltpu.SemaphoreType.DMA,
    ),
)
def _edge_kernel(y1_hbm, y2_hbm, row_hbm, col_hbm, z64_hbm, z16_hbm,
                 s1_out, s2_out, d1_out, d2_out,
                 acc_s1, acc_s2, acc_d1, acc_d2,
                 idx_r, idx_c, abuf, bbuf, hbuf, ones_buf,
                 sem_a, sem_b):
    cid = lax.axis_index("c")
    sid = lax.axis_index("s")
    wid = sid * NC + cid

    onehot = jnp.where(lax.iota(jnp.int32, 16) == 0,
                       jnp.float32(1.0), jnp.float32(0.0))

    def orow(i, carry):
        ones_buf[i, pl.ds(0, 16)] = onehot
        return carry

    lax.fori_loop(0, CH, orow, 0)

    base = sid * RPT
    pltpu.sync_copy(z64_hbm.at[pl.ds(base, RPT)], acc_s1.at[pl.ds(base, RPT)])
    pltpu.sync_copy(z64_hbm.at[pl.ds(base, RPT)], acc_s2.at[pl.ds(base, RPT)])
    pltpu.sync_copy(z16_hbm.at[pl.ds(base, RPT)], acc_d1.at[pl.ds(base, RPT)])
    pltpu.sync_copy(z16_hbm.at[pl.ds(base, RPT)], acc_d2.at[pl.ds(base, RPT)])
    plsc.subcore_barrier()

    ebase = wid * EPW

    def chunk(k, carry):
        off = ebase + k * CH
        pltpu.sync_copy(row_hbm.at[pl.ds(off, CH)], idx_r)
        pltpu.sync_copy(col_hbm.at[pl.ds(off, CH)], idx_c)
        cp_a = pltpu.async_copy(y1_hbm.at[idx_r], abuf, sem_a)
        cp_b = pltpu.async_copy(y2_hbm.at[idx_c], bbuf, sem_b)
        cp_a.wait()
        cp_b.wait()

        def erow(e, c2):
            for j in range(HIDDEN // 16):
                z = abuf[e, pl.ds(j * 16, 16)] + bbuf[e, pl.ds(j * 16, 16)]
                hbuf[e, pl.ds(j * 16, 16)] = z / (1.0 + jnp.exp(-z))
            return c2

        lax.fori_loop(0, CH, erow, 0)

        pltpu.sync_copy(hbuf, acc_s1.at[idx_r], add=True)
        pltpu.sync_copy(hbuf, acc_s2.at[idx_c], add=True)
        pltpu.sync_copy(ones_buf, acc_d1.at[idx_r], add=True)
        pltpu.sync_copy(ones_buf, acc_d2.at[idx_c], add=True)
        return carry

    lax.fori_loop(0, NCHUNK, chunk, 0)

    plsc.subcore_barrier()
    pltpu.sync_copy(acc_s1.at[pl.ds(base, RPT)], s1_out.at[cid, pl.ds(base, RPT)])
    pltpu.sync_copy(acc_s2.at[pl.ds(base, RPT)], s2_out.at[cid, pl.ds(base, RPT)])
    pltpu.sync_copy(acc_d1.at[pl.ds(base, RPT)], d1_out.at[cid, pl.ds(base, RPT)])
    pltpu.sync_copy(acc_d2.at[pl.ds(base, RPT)], d2_out.at[cid, pl.ds(base, RPT)])


def _combine_body(s1_ref, s2_ref, d1_ref, d2_ref, w2_ref, b2_ref, out_ref):
    s1 = s1_ref[0] + s1_ref[1]
    s2 = s2_ref[0] + s2_ref[1]
    d1 = d1_ref[0, :, 0:1] + d1_ref[1, :, 0:1]
    d2 = d2_ref[0, :, 0:1] + d2_ref[1, :, 0:1]
    w2 = w2_ref[...]
    out_ref[...] = (
        jnp.dot(s1, w2[:, 0:D], preferred_element_type=jnp.float32, precision=_HIGH)
        + jnp.dot(s2, w2[:, D:2 * D], preferred_element_type=jnp.float32,
                  precision=_HIGH)
        + d1 * b2_ref[0:1, 0:D]
        + d2 * b2_ref[0:1, D:2 * D]
    )


_combine = pl.pallas_call(
    _combine_body,
    grid=(NPAD // BNC,),
    in_specs=[
        pl.BlockSpec((NC, BNC, HIDDEN), lambda i: (0, i, 0)),
        pl.BlockSpec((NC, BNC, HIDDEN), lambda i: (0, i, 0)),
        pl.BlockSpec((NC, BNC, 16), lambda i: (0, i, 0)),
        pl.BlockSpec((NC, BNC, 16), lambda i: (0, i, 0)),
        pl.BlockSpec((HIDDEN, 2 * D), lambda i: (0, 0)),
        pl.BlockSpec((1, 2 * D), lambda i: (0, 0)),
    ],
    out_specs=pl.BlockSpec((BNC, D), lambda i: (i, 0)),
    out_shape=jax.ShapeDtypeStruct((NPAD, D), jnp.float32),
)


def kernel(x, edge_index, t, W1, b1, Wt, bt, W2, b2):
    half = TEMB // 2
    freqs = jnp.exp(
        -jnp.log(10000.0) * jnp.arange(half, dtype=jnp.float32) / (half - 1)
    )
    args = jnp.asarray(t, jnp.float32) * freqs
    temb = jnp.concatenate([jnp.sin(args), jnp.cos(args)], axis=-1)

    row = edge_index[0]
    col = edge_index[1]

    y1, y2 = _proj(
        x, W1, temb.reshape(1, TEMB), Wt, b1.reshape(1, HIDDEN),
        bt.reshape(1, HIDDEN)
    )
    z64 = jnp.zeros((NPAD, HIDDEN), jnp.float32)
    z16 = jnp.zeros((NPAD, 16), jnp.float32)
    s1, s2, d1, d2 = _edge_kernel(y1, y2, row, col, z64, z16)
    out = _combine(s1, s2, d1, d2, W2, b2.reshape(1, 2 * D))
    return out[:N]


# speedup vs baseline: 9.8809x; 1.2848x over previous
import functools

import jax
import jax.numpy as jnp
from jax import lax
from jax.experimental import pallas as pl
from jax.experimental.pallas import tpu as pltpu
from jax.experimental.pallas import tpu_sc as plsc

N = 10000
D = 128
E = 320000
HIDDEN = 64
TEMB = 128
AUG = 80

NC = 2
NS = 16
NW = NC * NS
EPW = E // NW
CH = 40
NCHUNK = EPW // CH
NPAD = 10240
RPT = NPAD // NS
BN = 1000
BNC = 1024

NIB = 4
NGB = 2
MAIN = NCHUNK - NIB - 2

_HIGH = lax.Precision.HIGHEST


def _proj_body(x_ref, w1_ref, temb_ref, wt_ref, b1_ref, bt_ref, y1_ref, y2_ref):
    cvec = (
        jnp.dot(temb_ref[...], wt_ref[...], preferred_element_type=jnp.float32,
                precision=_HIGH)
        + b1_ref[...]
        + bt_ref[...]
    )
    x = x_ref[...]
    y1_ref[...] = jnp.dot(x, w1_ref[0:D, :], preferred_element_type=jnp.float32,
                          precision=_HIGH) + cvec
    y2_ref[...] = jnp.dot(x, w1_ref[D:2 * D, :], preferred_element_type=jnp.float32,
                          precision=_HIGH)


_proj = pl.pallas_call(
    _proj_body,
    grid=(N // BN,),
    in_specs=[
        pl.BlockSpec((BN, D), lambda i: (i, 0)),
        pl.BlockSpec((2 * D, HIDDEN), lambda i: (0, 0)),
        pl.BlockSpec((1, TEMB), lambda i: (0, 0)),
        pl.BlockSpec((TEMB, HIDDEN), lambda i: (0, 0)),
        pl.BlockSpec((1, HIDDEN), lambda i: (0, 0)),
        pl.BlockSpec((1, HIDDEN), lambda i: (0, 0)),
    ],
    out_specs=[
        pl.BlockSpec((BN, HIDDEN), lambda i: (i, 0)),
        pl.BlockSpec((BN, HIDDEN), lambda i: (i, 0)),
    ],
    out_shape=[
        jax.ShapeDtypeStruct((N, HIDDEN), jnp.float32),
        jax.ShapeDtypeStruct((N, HIDDEN), jnp.float32),
    ],
)


_sc_mesh = plsc.VectorSubcoreMesh(
    core_axis_name="c", subcore_axis_name="s", num_cores=NC, num_subcores=NS
)


@functools.partial(
    pl.kernel,
    out_type=(
        jax.ShapeDtypeStruct((NC, NPAD, AUG), jnp.float32),
        jax.ShapeDtypeStruct((NC, NPAD, AUG), jnp.float32),
    ),
    mesh=_sc_mesh,
    compiler_params=pltpu.CompilerParams(use_tc_tiling_on_sc=False),
    scratch_types=(
        pltpu.VMEM_SHARED((NPAD, AUG), jnp.float32),
        pltpu.VMEM_SHARED((NPAD, AUG), jnp.float32),
        pltpu.VMEM((NIB, CH), jnp.int32),
        pltpu.VMEM((NIB, CH), jnp.int32),
        pltpu.VMEM((NGB, CH, HIDDEN), jnp.float32),
        pltpu.VMEM((NGB, CH, HIDDEN), jnp.float32),
        pltpu.VMEM((CH, AUG), jnp.float32),
        pltpu.SemaphoreType.DMA((NIB,)),
        pltpu.SemaphoreType.DMA((NGB,)),
    ),
)
def _edge_kernel(y1_hbm, y2_hbm, row_hbm, col_hbm, zacc_hbm,
                 s1_out, s2_out,
                 acc1, acc2,
                 idx_r, idx_c, abuf, bbuf, hbuf,
                 isem, gsem):
    cid = lax.axis_index("c")
    sid = lax.axis_index("s")
    wid = sid * NC + cid
    ebase = wid * EPW

    onehot = jnp.where(lax.iota(jnp.int32, 16) == 0,
                       jnp.float32(1.0), jnp.float32(0.0))

    @plsc.parallel_loop(0, CH, step=1, unroll=8)
    def _init_tail(e):
        hbuf[e, pl.ds(HIDDEN, 16)] = onehot

    base = sid * RPT
    pltpu.sync_copy(zacc_hbm.at[pl.ds(base, RPT)], acc1.at[pl.ds(base, RPT)])
    pltpu.sync_copy(zacc_hbm.at[pl.ds(base, RPT)], acc2.at[pl.ds(base, RPT)])
    plsc.subcore_barrier()

    def start_idx(m, q):
        off = ebase + m * CH
        pltpu.async_copy(row_hbm.at[pl.ds(off, CH)], idx_r.at[q], isem.at[q])
        pltpu.async_copy(col_hbm.at[pl.ds(off, CH)], idx_c.at[q], isem.at[q])

    def wait_idx(q):
        pltpu.make_async_copy(
            row_hbm.at[pl.ds(0, CH)], idx_r.at[q], isem.at[q]).wait()
        pltpu.make_async_copy(
            col_hbm.at[pl.ds(0, CH)], idx_c.at[q], isem.at[q]).wait()

    def start_gather(q, g):
        pltpu.async_copy(y1_hbm.at[idx_r.at[q]], abuf.at[g], gsem.at[g])
        pltpu.async_copy(y2_hbm.at[idx_c.at[q]], bbuf.at[g], gsem.at[g])

    def wait_gather(q, g):
        pltpu.make_async_copy(
            y1_hbm.at[idx_r.at[q]], abuf.at[g], gsem.at[g]).wait()
        pltpu.make_async_copy(
            y2_hbm.at[idx_c.at[q]], bbuf.at[g], gsem.at[g]).wait()

    def process(q, g):
        wait_gather(q, g)
        a = abuf.at[g]
        b = bbuf.at[g]

        @plsc.parallel_loop(0, CH, step=1, unroll=8)
        def _silu_row(e):
            for j in range(HIDDEN // 16):
                z = a[e, pl.ds(j * 16, 16)] + b[e, pl.ds(j * 16, 16)]
                hbuf[e, pl.ds(j * 16, 16)] = z / (1.0 + jnp.exp(-z))

        pltpu.sync_copy(hbuf, acc1.at[idx_r.at[q]], add=True)
        pltpu.sync_copy(hbuf, acc2.at[idx_c.at[q]], add=True)

    for m in range(NIB):
        start_idx(m, m)
    wait_idx(0)
    start_gather(0, 0)

    @pl.loop(0, MAIN, step=NIB)
    def _quad(k):
        for d in range(NIB):
            m = k + d
            qn = (d + 1) % NIB
            wait_idx(qn)
            start_gather(qn, (d + 1) % NGB)
            process(d, d % NGB)
            start_idx(m + NIB, d)

    for m in range(MAIN, NCHUNK):
        if m + 1 < NCHUNK:
            wait_idx((m + 1) % NIB)
            start_gather((m + 1) % NIB, (m + 1) % NGB)
        process(m % NIB, m % NGB)
        if m + NIB < NCHUNK:
            start_idx(m + NIB, m % NIB)

    plsc.subcore_barrier()
    pltpu.sync_copy(acc1.at[pl.ds(base, RPT)], s1_out.at[cid, pl.ds(base, RPT)])
    pltpu.sync_copy(acc2.at[pl.ds(base, RPT)], s2_out.at[cid, pl.ds(base, RPT)])


def _combine_body(s1_ref, s2_ref, w2a1_ref, w2a2_ref, out_ref):
    s1 = s1_ref[0] + s1_ref[1]
    s2 = s2_ref[0] + s2_ref[1]
    out_ref[...] = (
        jnp.dot(s1, w2a1_ref[...], preferred_element_type=jnp.float32,
                precision=_HIGH)
        + jnp.dot(s2, w2a2_ref[...], preferred_element_type=jnp.float32,
                  precision=_HIGH)
    )


_combine = pl.pallas_call(
    _combine_body,
    grid=(NPAD // BNC,),
    in_specs=[
        pl.BlockSpec((NC, BNC, AUG), lambda i: (0, i, 0)),
        pl.BlockSpec((NC, BNC, AUG), lambda i: (0, i, 0)),
        pl.BlockSpec((AUG, D), lambda i: (0, 0)),
        pl.BlockSpec((AUG, D), lambda i: (0, 0)),
    ],
    out_specs=pl.BlockSpec((BNC, D), lambda i: (i, 0)),
    out_shape=jax.ShapeDtypeStruct((NPAD, D), jnp.float32),
)


def kernel(x, edge_index, t, W1, b1, Wt, bt, W2, b2):
    half = TEMB // 2
    freqs = jnp.exp(
        -jnp.log(10000.0) * jnp.arange(half, dtype=jnp.float32) / (half - 1)
    )
    args = jnp.asarray(t, jnp.float32) * freqs
    temb = jnp.concatenate([jnp.sin(args), jnp.cos(args)], axis=-1)

    row = edge_index[0]
    col = edge_index[1]

    y1, y2 = _proj(
        x, W1, temb.reshape(1, TEMB), Wt, b1.reshape(1, HIDDEN),
        bt.reshape(1, HIDDEN)
    )
    zacc = jnp.zeros((NPAD, AUG), jnp.float32)
    s1, s2 = _edge_kernel(y1, y2, row, col, zacc)

    pad = jnp.zeros((AUG - HIDDEN - 1, D), jnp.float32)
    w2a1 = jnp.concatenate([W2[:, :D], b2[:D].reshape(1, D), pad], axis=0)
    w2a2 = jnp.concatenate([W2[:, D:], b2[D:].reshape(1, D), pad], axis=0)

    out = _combine(s1, s2, w2a1, w2a2)
    return out[:N]


# speedup vs baseline: 11.1256x; 1.1260x over previous
import functools

import jax
import jax.numpy as jnp
from jax import lax
from jax.experimental import pallas as pl
from jax.experimental.pallas import tpu as pltpu
from jax.experimental.pallas import tpu_sc as plsc

N = 10000
D = 128
E = 320000
HIDDEN = 64
TEMB = 128
AUG = 80

NC = 2
NS = 16
NW = NC * NS
EPW = E // NW
CH = 40
NCHUNK = EPW // CH
NPAD = 10240
RPT = NPAD // NS
BN = 1000
BNC = 1024

NIB = 6
NGB = 2
NHB = 2
IPD = NIB - 2
MAIN = 240

_HIGH = lax.Precision.HIGHEST


def _proj_body(x_ref, w1_ref, temb_ref, wt_ref, b1_ref, bt_ref, y1_ref, y2_ref):
    cvec = (
        jnp.dot(temb_ref[...], wt_ref[...], preferred_element_type=jnp.float32,
                precision=_HIGH)
        + b1_ref[...]
        + bt_ref[...]
    )
    x = x_ref[...]
    y1_ref[...] = jnp.dot(x, w1_ref[0:D, :], preferred_element_type=jnp.float32,
                          precision=_HIGH) + cvec
    y2_ref[...] = jnp.dot(x, w1_ref[D:2 * D, :], preferred_element_type=jnp.float32,
                          precision=_HIGH)


_proj = pl.pallas_call(
    _proj_body,
    grid=(N // BN,),
    in_specs=[
        pl.BlockSpec((BN, D), lambda i: (i, 0)),
        pl.BlockSpec((2 * D, HIDDEN), lambda i: (0, 0)),
        pl.BlockSpec((1, TEMB), lambda i: (0, 0)),
        pl.BlockSpec((TEMB, HIDDEN), lambda i: (0, 0)),
        pl.BlockSpec((1, HIDDEN), lambda i: (0, 0)),
        pl.BlockSpec((1, HIDDEN), lambda i: (0, 0)),
    ],
    out_specs=[
        pl.BlockSpec((BN, HIDDEN), lambda i: (i, 0)),
        pl.BlockSpec((BN, HIDDEN), lambda i: (i, 0)),
    ],
    out_shape=[
        jax.ShapeDtypeStruct((N, HIDDEN), jnp.float32),
        jax.ShapeDtypeStruct((N, HIDDEN), jnp.float32),
    ],
)


_sc_mesh = plsc.VectorSubcoreMesh(
    core_axis_name="c", subcore_axis_name="s", num_cores=NC, num_subcores=NS
)


@functools.partial(
    pl.kernel,
    out_type=(
        jax.ShapeDtypeStruct((NC, NPAD, AUG), jnp.float32),
        jax.ShapeDtypeStruct((NC, NPAD, AUG), jnp.float32),
    ),
    mesh=_sc_mesh,
    compiler_params=pltpu.CompilerParams(use_tc_tiling_on_sc=False),
    scratch_types=(
        pltpu.VMEM_SHARED((NPAD, AUG), jnp.float32),
        pltpu.VMEM_SHARED((NPAD, AUG), jnp.float32),
        pltpu.VMEM((NIB, CH), jnp.int32),
        pltpu.VMEM((NIB, CH), jnp.int32),
        pltpu.VMEM((NGB, CH, HIDDEN), jnp.float32),
        pltpu.VMEM((NGB, CH, HIDDEN), jnp.float32),
        pltpu.VMEM((NHB, CH, AUG), jnp.float32),
        pltpu.SemaphoreType.DMA((NIB,)),
        pltpu.SemaphoreType.DMA((NGB,)),
        pltpu.SemaphoreType.DMA((NHB,)),
    ),
)
def _edge_kernel(y1_hbm, y2_hbm, row_hbm, col_hbm, zacc_hbm,
                 s1_out, s2_out,
                 acc1, acc2,
                 idx_r, idx_c, abuf, bbuf, hbuf,
                 isem, gsem, ssem):
    cid = lax.axis_index("c")
    sid = lax.axis_index("s")
    wid = sid * NC + cid
    ebase = wid * EPW

    onehot = jnp.where(lax.iota(jnp.int32, 16) == 0,
                       jnp.float32(1.0), jnp.float32(0.0))

    @plsc.parallel_loop(0, NHB * CH, step=1, unroll=8)
    def _init_tail(e):
        hbuf[e // CH, e % CH, pl.ds(HIDDEN, 16)] = onehot

    base = sid * RPT
    pltpu.sync_copy(zacc_hbm.at[pl.ds(base, RPT)], acc1.at[pl.ds(base, RPT)])
    pltpu.sync_copy(zacc_hbm.at[pl.ds(base, RPT)], acc2.at[pl.ds(base, RPT)])
    plsc.subcore_barrier()

    def start_idx(m, q):
        off = ebase + m * CH
        pltpu.async_copy(row_hbm.at[pl.ds(off, CH)], idx_r.at[q], isem.at[q])
        pltpu.async_copy(col_hbm.at[pl.ds(off, CH)], idx_c.at[q], isem.at[q])

    def wait_idx(q):
        pltpu.make_async_copy(
            row_hbm.at[pl.ds(0, CH)], idx_r.at[q], isem.at[q]).wait()
        pltpu.make_async_copy(
            col_hbm.at[pl.ds(0, CH)], idx_c.at[q], isem.at[q]).wait()

    def start_gather(q, g):
        pltpu.async_copy(y1_hbm.at[idx_r.at[q]], abuf.at[g], gsem.at[g])
        pltpu.async_copy(y2_hbm.at[idx_c.at[q]], bbuf.at[g], gsem.at[g])

    def wait_gather(q, g):
        pltpu.make_async_copy(
            y1_hbm.at[idx_r.at[q]], abuf.at[g], gsem.at[g]).wait()
        pltpu.make_async_copy(
            y2_hbm.at[idx_c.at[q]], bbuf.at[g], gsem.at[g]).wait()

    def silu(q, g, h):
        a = abuf.at[g]
        b = bbuf.at[g]
        hb = hbuf.at[h]

        @plsc.parallel_loop(0, CH, step=1, unroll=8)
        def _silu_row(e):
            for j in range(HIDDEN // 16):
                z = a[e, pl.ds(j * 16, 16)] + b[e, pl.ds(j * 16, 16)]
                hb[e, pl.ds(j * 16, 16)] = z / (1.0 + jnp.exp(-z))

        pltpu.async_copy(hb, acc1.at[idx_r.at[q]], ssem.at[h], add=True)
        pltpu.async_copy(hb, acc2.at[idx_c.at[q]], ssem.at[h], add=True)

    def wait_scatter(q, h):
        pltpu.make_async_copy(
            hbuf.at[h], acc1.at[idx_r.at[q]], ssem.at[h]).wait()
        pltpu.make_async_copy(
            hbuf.at[h], acc2.at[idx_c.at[q]], ssem.at[h]).wait()

    for m in range(IPD):
        start_idx(m, m)
    wait_idx(0)
    start_gather(0, 0)

    for m in (0, 1):
        wait_idx(m + 1)
        start_gather(m + 1, (m + 1) % NGB)
        wait_gather(m, m % NGB)
        silu(m, m % NGB, m % NHB)
        start_idx(m + IPD, (m + IPD) % NIB)

    @pl.loop(2, 2 + MAIN, step=NIB)
    def _six(k):
        for d in range(NIB):
            m = k + d
            q = (2 + d) % NIB
            g = d % NGB
            h = d % NHB
            wait_idx((q + 1) % NIB)
            start_gather((q + 1) % NIB, (g + 1) % NGB)
            wait_scatter((q - 2) % NIB, h)
            wait_gather(q, g)
            silu(q, g, h)
            start_idx(m + IPD, (q - 2) % NIB)

    for m in range(2 + MAIN, NCHUNK):
        if m + 1 < NCHUNK:
            wait_idx((m + 1) % NIB)
            start_gather((m + 1) % NIB, (m + 1) % NGB)
        wait_scatter((m - 2) % NIB, m % NHB)
        wait_gather(m % NIB, m % NGB)
        silu(m % NIB, m % NGB, m % NHB)
        if m + IPD < NCHUNK:
            start_idx(m + IPD, (m - 2) % NIB)

    wait_scatter((NCHUNK - 2) % NIB, (NCHUNK - 2) % NHB)
    wait_scatter((NCHUNK - 1) % NIB, (NCHUNK - 1) % NHB)

    plsc.subcore_barrier()
    pltpu.sync_copy(acc1.at[pl.ds(base, RPT)], s1_out.at[cid, pl.ds(base, RPT)])
    pltpu.sync_copy(acc2.at[pl.ds(base, RPT)], s2_out.at[cid, pl.ds(base, RPT)])


def _combine_body(s1_ref, s2_ref, w2a1_ref, w2a2_ref, out_ref):
    s1 = s1_ref[0] + s1_ref[1]
    s2 = s2_ref[0] + s2_ref[1]
    out_ref[...] = (
        jnp.dot(s1, w2a1_ref[...], preferred_element_type=jnp.float32,
                precision=_HIGH)
        + jnp.dot(s2, w2a2_ref[...], preferred_element_type=jnp.float32,
                  precision=_HIGH)
    )


_combine = pl.pallas_call(
    _combine_body,
    grid=(NPAD // BNC,),
    in_specs=[
        pl.BlockSpec((NC, BNC, AUG), lambda i: (0, i, 0)),
        pl.BlockSpec((NC, BNC, AUG), lambda i: (0, i, 0)),
        pl.BlockSpec((AUG, D), lambda i: (0, 0)),
        pl.BlockSpec((AUG, D), lambda i: (0, 0)),
    ],
    out_specs=pl.BlockSpec((BNC, D), lambda i: (i, 0)),
    out_shape=jax.ShapeDtypeStruct((NPAD, D), jnp.float32),
)


def kernel(x, edge_index, t, W1, b1, Wt, bt, W2, b2):
    half = TEMB // 2
    freqs = jnp.exp(
        -jnp.log(10000.0) * jnp.arange(half, dtype=jnp.float32) / (half - 1)
    )
    args = jnp.asarray(t, jnp.float32) * freqs
    temb = jnp.concatenate([jnp.sin(args), jnp.cos(args)], axis=-1)

    row = edge_index[0]
    col = edge_index[1]

    y1, y2 = _proj(
        x, W1, temb.reshape(1, TEMB), Wt, b1.reshape(1, HIDDEN),
        bt.reshape(1, HIDDEN)
    )
    zacc = jnp.zeros((NPAD, AUG), jnp.float32)
    s1, s2 = _edge_kernel(y1, y2, row, col, zacc)

    pad = jnp.zeros((AUG - HIDDEN - 1, D), jnp.float32)
    w2a1 = jnp.concatenate([W2[:, :D], b2[:D].reshape(1, D), pad], axis=0)
    w2a2 = jnp.concatenate([W2[:, D:], b2[D:].reshape(1, D), pad], axis=0)

    out = _combine(s1, s2, w2a1, w2a2)
    return out[:N]


# speedup vs baseline: 11.2067x; 1.0073x over previous
import functools

import jax
import jax.numpy as jnp
from jax import lax
from jax.experimental import pallas as pl
from jax.experimental.pallas import tpu as pltpu
from jax.experimental.pallas import tpu_sc as plsc

N = 10000
D = 128
E = 320000
HIDDEN = 64
TEMB = 128
AUG = 80

NC = 2
NS = 16
NW = NC * NS
EPW = E // NW
CH = 40
NCHUNK = EPW // CH
NPAD = 10240
RPT = NPAD // NS
BN = 1000
BNC = 1000

NIB = 6
NGB = 2
NHB = 2
IPD = NIB - 2
MAIN = 240

_HIGH = lax.Precision.HIGHEST


def _proj_body(x_ref, w1_ref, t_ref, wt_ref, b1_ref, bt_ref, y1_ref, y2_ref):
    half = TEMB // 2
    k = lax.iota(jnp.int32, half).astype(jnp.float32)
    freqs = jnp.exp(-jnp.log(10000.0) * k / (half - 1)).reshape(1, half)
    args = t_ref[0, 0] * freqs
    temb = jnp.concatenate([jnp.sin(args), jnp.cos(args)], axis=-1)
    cvec = (
        jnp.dot(temb, wt_ref[...], preferred_element_type=jnp.float32,
                precision=_HIGH)
        + b1_ref[...]
        + bt_ref[...]
    )
    x = x_ref[...]
    y1_ref[...] = jnp.dot(x, w1_ref[0:D, :], preferred_element_type=jnp.float32,
                          precision=_HIGH) + cvec
    y2_ref[...] = jnp.dot(x, w1_ref[D:2 * D, :], preferred_element_type=jnp.float32,
                          precision=_HIGH)


_proj = pl.pallas_call(
    _proj_body,
    grid=(N // BN,),
    in_specs=[
        pl.BlockSpec((BN, D), lambda i: (i, 0)),
        pl.BlockSpec((2 * D, HIDDEN), lambda i: (0, 0)),
        pl.BlockSpec((1, 1), lambda i: (0, 0)),
        pl.BlockSpec((TEMB, HIDDEN), lambda i: (0, 0)),
        pl.BlockSpec((1, HIDDEN), lambda i: (0, 0)),
        pl.BlockSpec((1, HIDDEN), lambda i: (0, 0)),
    ],
    out_specs=[
        pl.BlockSpec((BN, HIDDEN), lambda i: (i, 0)),
        pl.BlockSpec((BN, HIDDEN), lambda i: (i, 0)),
    ],
    out_shape=[
        jax.ShapeDtypeStruct((N, HIDDEN), jnp.float32),
        jax.ShapeDtypeStruct((N, HIDDEN), jnp.float32),
    ],
)


_sc_mesh = plsc.VectorSubcoreMesh(
    core_axis_name="c", subcore_axis_name="s", num_cores=NC, num_subcores=NS
)


@functools.partial(
    pl.kernel,
    out_type=(
        jax.ShapeDtypeStruct((NC, NPAD, AUG), jnp.float32),
        jax.ShapeDtypeStruct((NC, NPAD, AUG), jnp.float32),
    ),
    mesh=_sc_mesh,
    compiler_params=pltpu.CompilerParams(use_tc_tiling_on_sc=False),
    scratch_types=(
        pltpu.VMEM_SHARED((NPAD, AUG), jnp.float32),
        pltpu.VMEM_SHARED((NPAD, AUG), jnp.float32),
        pltpu.VMEM((NIB, CH), jnp.int32),
        pltpu.VMEM((NIB, CH), jnp.int32),
        pltpu.VMEM((NGB, CH, HIDDEN), jnp.float32),
        pltpu.VMEM((NGB, CH, HIDDEN), jnp.float32),
        pltpu.VMEM((NHB, CH, AUG), jnp.float32),
        pltpu.SemaphoreType.DMA((NIB,)),
        pltpu.SemaphoreType.DMA((NGB,)),
        pltpu.SemaphoreType.DMA((NHB,)),
    ),
)
def _edge_kernel(y1_hbm, y2_hbm, row_hbm, col_hbm, zacc_hbm,
                 s1_out, s2_out,
                 acc1, acc2,
                 idx_r, idx_c, abuf, bbuf, hbuf,
                 isem, gsem, ssem):
    cid = lax.axis_index("c")
    sid = lax.axis_index("s")
    wid = sid * NC + cid
    ebase = wid * EPW

    onehot = jnp.where(lax.iota(jnp.int32, 16) == 0,
                       jnp.float32(1.0), jnp.float32(0.0))

    @plsc.parallel_loop(0, NHB * CH, step=1, unroll=8)
    def _init_tail(e):
        hbuf[e // CH, e % CH, pl.ds(HIDDEN, 16)] = onehot

    base = sid * RPT
    pltpu.sync_copy(zacc_hbm.at[pl.ds(base, RPT)], acc1.at[pl.ds(base, RPT)])
    pltpu.sync_copy(zacc_hbm.at[pl.ds(base, RPT)], acc2.at[pl.ds(base, RPT)])
    plsc.subcore_barrier()

    def start_idx(m, q):
        off = ebase + m * CH
        pltpu.async_copy(row_hbm.at[pl.ds(off, CH)], idx_r.at[q], isem.at[q])
        pltpu.async_copy(col_hbm.at[pl.ds(off, CH)], idx_c.at[q], isem.at[q])

    def wait_idx(q):
        pltpu.make_async_copy(
            row_hbm.at[pl.ds(0, CH)], idx_r.at[q], isem.at[q]).wait()
        pltpu.make_async_copy(
            col_hbm.at[pl.ds(0, CH)], idx_c.at[q], isem.at[q]).wait()

    def start_gather(q, g):
        pltpu.async_copy(y1_hbm.at[idx_r.at[q]], abuf.at[g], gsem.at[g])
        pltpu.async_copy(y2_hbm.at[idx_c.at[q]], bbuf.at[g], gsem.at[g])

    def wait_gather(q, g):
        pltpu.make_async_copy(
            y1_hbm.at[idx_r.at[q]], abuf.at[g], gsem.at[g]).wait()
        pltpu.make_async_copy(
            y2_hbm.at[idx_c.at[q]], bbuf.at[g], gsem.at[g]).wait()

    def silu(q, g, h):
        a = abuf.at[g]
        b = bbuf.at[g]
        hb = hbuf.at[h]

        @plsc.parallel_loop(0, CH, step=1, unroll=8)
        def _silu_row(e):
            for j in range(HIDDEN // 16):
                z = a[e, pl.ds(j * 16, 16)] + b[e, pl.ds(j * 16, 16)]
                hb[e, pl.ds(j * 16, 16)] = z / (1.0 + jnp.exp(-z))

        pltpu.async_copy(hb, acc1.at[idx_r.at[q]], ssem.at[h], add=True)
        pltpu.async_copy(hb, acc2.at[idx_c.at[q]], ssem.at[h], add=True)

    def wait_scatter(q, h):
        pltpu.make_async_copy(
            hbuf.at[h], acc1.at[idx_r.at[q]], ssem.at[h]).wait()
        pltpu.make_async_copy(
            hbuf.at[h], acc2.at[idx_c.at[q]], ssem.at[h]).wait()

    for m in range(IPD):
        start_idx(m, m)
    wait_idx(0)
    start_gather(0, 0)

    for m in (0, 1):
        wait_idx(m + 1)
        start_gather(m + 1, (m + 1) % NGB)
        wait_gather(m, m % NGB)
        silu(m, m % NGB, m % NHB)
        start_idx(m + IPD, (m + IPD) % NIB)

    @pl.loop(2, 2 + MAIN, step=NIB)
    def _six(k):
        for d in range(NIB):
            m = k + d
            q = (2 + d) % NIB
            g = d % NGB
            h = d % NHB
            wait_idx((q + 1) % NIB)
            start_gather((q + 1) % NIB, (g + 1) % NGB)
            wait_scatter((q - 2) % NIB, h)
            wait_gather(q, g)
            silu(q, g, h)
            start_idx(m + IPD, (q - 2) % NIB)

    for m in range(2 + MAIN, NCHUNK):
        if m + 1 < NCHUNK:
            wait_idx((m + 1) % NIB)
            start_gather((m + 1) % NIB, (m + 1) % NGB)
        wait_scatter((m - 2) % NIB, m % NHB)
        wait_gather(m % NIB, m % NGB)
        silu(m % NIB, m % NGB, m % NHB)
        if m + IPD < NCHUNK:
            start_idx(m + IPD, (m - 2) % NIB)

    wait_scatter((NCHUNK - 2) % NIB, (NCHUNK - 2) % NHB)
    wait_scatter((NCHUNK - 1) % NIB, (NCHUNK - 1) % NHB)

    plsc.subcore_barrier()
    pltpu.sync_copy(acc1.at[pl.ds(base, RPT)], s1_out.at[cid, pl.ds(base, RPT)])
    pltpu.sync_copy(acc2.at[pl.ds(base, RPT)], s2_out.at[cid, pl.ds(base, RPT)])


def _combine_body(s1_ref, s2_ref, w2_ref, b2_ref, out_ref):
    s1 = s1_ref[0] + s1_ref[1]
    s2 = s2_ref[0] + s2_ref[1]
    w2 = w2_ref[...]
    out_ref[...] = (
        jnp.dot(s1[:, 0:HIDDEN], w2[:, 0:D],
                preferred_element_type=jnp.float32, precision=_HIGH)
        + jnp.dot(s2[:, 0:HIDDEN], w2[:, D:2 * D],
                  preferred_element_type=jnp.float32, precision=_HIGH)
        + s1[:, HIDDEN:HIDDEN + 1] * b2_ref[0:1, 0:D]
        + s2[:, HIDDEN:HIDDEN + 1] * b2_ref[0:1, D:2 * D]
    )


_combine = pl.pallas_call(
    _combine_body,
    grid=(N // BNC,),
    in_specs=[
        pl.BlockSpec((NC, BNC, AUG), lambda i: (0, i, 0)),
        pl.BlockSpec((NC, BNC, AUG), lambda i: (0, i, 0)),
        pl.BlockSpec((HIDDEN, 2 * D), lambda i: (0, 0)),
        pl.BlockSpec((1, 2 * D), lambda i: (0, 0)),
    ],
    out_specs=pl.BlockSpec((BNC, D), lambda i: (i, 0)),
    out_shape=jax.ShapeDtypeStruct((N, D), jnp.float32),
)


def kernel(x, edge_index, t, W1, b1, Wt, bt, W2, b2):
    row = edge_index[0]
    col = edge_index[1]

    y1, y2 = _proj(
        x, W1, jnp.asarray(t, jnp.float32).reshape(1, 1), Wt,
        b1.reshape(1, HIDDEN), bt.reshape(1, HIDDEN)
    )
    zacc = jnp.zeros((NPAD, AUG), jnp.float32)
    s1, s2 = _edge_kernel(y1, y2, row, col, zacc)
    return _combine(s1, s2, W2, b2.reshape(1, 2 * D))


# speedup vs baseline: 11.4680x; 1.0233x over previous
import functools

import jax
import jax.numpy as jnp
from jax import lax
from jax.experimental import pallas as pl
from jax.experimental.pallas import tpu as pltpu
from jax.experimental.pallas import tpu_sc as plsc

N = 10000
D = 128
E = 320000
HIDDEN = 64
TEMB = 128
AUG = 80

NC = 2
NS = 16
NW = NC * NS
EPW = E // NW
CH = 40
NCHUNK = EPW // CH
NPAD = 10240
RPT = NPAD // NS
BN = 1000
BNC = 1000

NIB = 6
NGB = 2
NHB = 2
IPD = NIB - 2
MAIN = 240

_HIGH = lax.Precision.HIGHEST


def _proj_body(x_ref, w1_ref, t_ref, wt_ref, b1_ref, bt_ref, y1_ref, y2_ref):
    half = TEMB // 2
    k = lax.iota(jnp.int32, half).astype(jnp.float32)
    freqs = jnp.exp(-jnp.log(10000.0) * k / (half - 1)).reshape(1, half)
    args = t_ref[0, 0] * freqs
    temb = jnp.concatenate([jnp.sin(args), jnp.cos(args)], axis=-1)
    cvec = (
        jnp.dot(temb, wt_ref[...], preferred_element_type=jnp.float32,
                precision=_HIGH)
        + b1_ref[...]
        + bt_ref[...]
    )
    x = x_ref[...]
    y1_ref[...] = (jnp.dot(x, w1_ref[0:D, :], preferred_element_type=jnp.float32,
                           precision=_HIGH) + cvec).astype(jnp.bfloat16)
    y2_ref[...] = jnp.dot(x, w1_ref[D:2 * D, :], preferred_element_type=jnp.float32,
                          precision=_HIGH).astype(jnp.bfloat16)


_proj = pl.pallas_call(
    _proj_body,
    grid=(N // BN,),
    in_specs=[
        pl.BlockSpec((BN, D), lambda i: (i, 0)),
        pl.BlockSpec((2 * D, HIDDEN), lambda i: (0, 0)),
        pl.BlockSpec((1, 1), lambda i: (0, 0)),
        pl.BlockSpec((TEMB, HIDDEN), lambda i: (0, 0)),
        pl.BlockSpec((1, HIDDEN), lambda i: (0, 0)),
        pl.BlockSpec((1, HIDDEN), lambda i: (0, 0)),
    ],
    out_specs=[
        pl.BlockSpec((BN, HIDDEN), lambda i: (i, 0)),
        pl.BlockSpec((BN, HIDDEN), lambda i: (i, 0)),
    ],
    out_shape=[
        jax.ShapeDtypeStruct((N, HIDDEN), jnp.bfloat16),
        jax.ShapeDtypeStruct((N, HIDDEN), jnp.bfloat16),
    ],
)


_sc_mesh = plsc.VectorSubcoreMesh(
    core_axis_name="c", subcore_axis_name="s", num_cores=NC, num_subcores=NS
)


@functools.partial(
    pl.kernel,
    out_type=(
        jax.ShapeDtypeStruct((NC, NPAD, AUG), jnp.float32),
        jax.ShapeDtypeStruct((NC, NPAD, AUG), jnp.float32),
    ),
    mesh=_sc_mesh,
    compiler_params=pltpu.CompilerParams(use_tc_tiling_on_sc=False),
    scratch_types=(
        pltpu.VMEM_SHARED((NPAD, AUG), jnp.float32),
        pltpu.VMEM_SHARED((NPAD, AUG), jnp.float32),
        pltpu.VMEM((NIB, CH), jnp.int32),
        pltpu.VMEM((NIB, CH), jnp.int32),
        pltpu.VMEM((NGB, CH, HIDDEN // 2), jnp.int32),
        pltpu.VMEM((NGB, CH, HIDDEN // 2), jnp.int32),
        pltpu.VMEM((NHB, CH, AUG), jnp.float32),
        pltpu.SemaphoreType.DMA((NIB,)),
        pltpu.SemaphoreType.DMA((NGB,)),
        pltpu.SemaphoreType.DMA((NHB,)),
    ),
)
def _edge_kernel(y1_hbm, y2_hbm, row_hbm, col_hbm, zacc_hbm,
                 s1_out, s2_out,
                 acc1, acc2,
                 idx_r, idx_c, abuf, bbuf, hbuf,
                 isem, gsem, ssem):
    cid = lax.axis_index("c")
    sid = lax.axis_index("s")
    wid = sid * NC + cid
    ebase = wid * EPW

    onehot = jnp.where(lax.iota(jnp.int32, 16) == 0,
                       jnp.float32(1.0), jnp.float32(0.0))

    @plsc.parallel_loop(0, NHB * CH, step=1, unroll=8)
    def _init_tail(e):
        hbuf[e // CH, e % CH, pl.ds(HIDDEN, 16)] = onehot

    base = sid * RPT
    pltpu.sync_copy(zacc_hbm.at[pl.ds(base, RPT)], acc1.at[pl.ds(base, RPT)])
    pltpu.sync_copy(zacc_hbm.at[pl.ds(base, RPT)], acc2.at[pl.ds(base, RPT)])
    plsc.subcore_barrier()

    def start_idx(m, q):
        off = ebase + m * CH
        pltpu.async_copy(row_hbm.at[pl.ds(off, CH)], idx_r.at[q], isem.at[q])
        pltpu.async_copy(col_hbm.at[pl.ds(off, CH)], idx_c.at[q], isem.at[q])

    def wait_idx(q):
        pltpu.make_async_copy(
            row_hbm.at[pl.ds(0, CH)], idx_r.at[q], isem.at[q]).wait()
        pltpu.make_async_copy(
            col_hbm.at[pl.ds(0, CH)], idx_c.at[q], isem.at[q]).wait()

    def start_gather(q, g):
        pltpu.async_copy(y1_hbm.at[idx_r.at[q]], abuf.at[g], gsem.at[g])
        pltpu.async_copy(y2_hbm.at[idx_c.at[q]], bbuf.at[g], gsem.at[g])

    def wait_gather(q, g):
        pltpu.make_async_copy(
            y1_hbm.at[idx_r.at[q]], abuf.at[g], gsem.at[g]).wait()
        pltpu.make_async_copy(
            y2_hbm.at[idx_c.at[q]], bbuf.at[g], gsem.at[g]).wait()

    def silu(q, g, h):
        a = abuf.at[g]
        b = bbuf.at[g]
        hb = hbuf.at[h]

        mask = jnp.int32(-65536)

        @plsc.parallel_loop(0, CH, step=1, unroll=8)
        def _silu_row(e):
            for j in range(HIDDEN // 32):
                wa = a[e, pl.ds(j * 16, 16)]
                wb = b[e, pl.ds(j * 16, 16)]
                al = lax.bitcast_convert_type(wa << 16, jnp.float32)
                au = lax.bitcast_convert_type(wa & mask, jnp.float32)
                bl = lax.bitcast_convert_type(wb << 16, jnp.float32)
                bu = lax.bitcast_convert_type(wb & mask, jnp.float32)
                zl = al + bl
                zu = au + bu
                hb[e, pl.ds(j * 32, 16)] = zl / (1.0 + jnp.exp(-zl))
                hb[e, pl.ds(j * 32 + 16, 16)] = zu / (1.0 + jnp.exp(-zu))

        pltpu.async_copy(hb, acc1.at[idx_r.at[q]], ssem.at[h], add=True)
        pltpu.async_copy(hb, acc2.at[idx_c.at[q]], ssem.at[h], add=True)

    def wait_scatter(q, h):
        pltpu.make_async_copy(
            hbuf.at[h], acc1.at[idx_r.at[q]], ssem.at[h]).wait()
        pltpu.make_async_copy(
            hbuf.at[h], acc2.at[idx_c.at[q]], ssem.at[h]).wait()

    for m in range(IPD):
        start_idx(m, m)
    wait_idx(0)
    start_gather(0, 0)

    for m in (0, 1):
        wait_idx(m + 1)
        start_gather(m + 1, (m + 1) % NGB)
        wait_gather(m, m % NGB)
        silu(m, m % NGB, m % NHB)
        start_idx(m + IPD, (m + IPD) % NIB)

    @pl.loop(2, 2 + MAIN, step=NIB)
    def _six(k):
        for d in range(NIB):
            m = k + d
            q = (2 + d) % NIB
            g = d % NGB
            h = d % NHB
            wait_idx((q + 1) % NIB)
            start_gather((q + 1) % NIB, (g + 1) % NGB)
            wait_scatter((q - 2) % NIB, h)
            wait_gather(q, g)
            silu(q, g, h)
            start_idx(m + IPD, (q - 2) % NIB)

    for m in range(2 + MAIN, NCHUNK):
        if m + 1 < NCHUNK:
            wait_idx((m + 1) % NIB)
            start_gather((m + 1) % NIB, (m + 1) % NGB)
        wait_scatter((m - 2) % NIB, m % NHB)
        wait_gather(m % NIB, m % NGB)
        silu(m % NIB, m % NGB, m % NHB)
        if m + IPD < NCHUNK:
            start_idx(m + IPD, (m - 2) % NIB)

    wait_scatter((NCHUNK - 2) % NIB, (NCHUNK - 2) % NHB)
    wait_scatter((NCHUNK - 1) % NIB, (NCHUNK - 1) % NHB)

    plsc.subcore_barrier()
    pltpu.sync_copy(acc1.at[pl.ds(base, RPT)], s1_out.at[cid, pl.ds(base, RPT)])
    pltpu.sync_copy(acc2.at[pl.ds(base, RPT)], s2_out.at[cid, pl.ds(base, RPT)])


def _combine_body(s1_ref, s2_ref, w2_ref, b2_ref, out_ref):
    s1 = s1_ref[0] + s1_ref[1]
    s2 = s2_ref[0] + s2_ref[1]
    w2 = w2_ref[...]
    out_ref[...] = (
        jnp.dot(s1[:, 0:HIDDEN], w2[:, 0:D],
                preferred_element_type=jnp.float32, precision=_HIGH)
        + jnp.dot(s2[:, 0:HIDDEN], w2[:, D:2 * D],
                  preferred_element_type=jnp.float32, precision=_HIGH)
        + s1[:, HIDDEN:HIDDEN + 1] * b2_ref[0:1, 0:D]
        + s2[:, HIDDEN:HIDDEN + 1] * b2_ref[0:1, D:2 * D]
    )


_combine = pl.pallas_call(
    _combine_body,
    grid=(N // BNC,),
    in_specs=[
        pl.BlockSpec((NC, BNC, AUG), lambda i: (0, i, 0)),
        pl.BlockSpec((NC, BNC, AUG), lambda i: (0, i, 0)),
        pl.BlockSpec((HIDDEN, 2 * D), lambda i: (0, 0)),
        pl.BlockSpec((1, 2 * D), lambda i: (0, 0)),
    ],
    out_specs=pl.BlockSpec((BNC, D), lambda i: (i, 0)),
    out_shape=jax.ShapeDtypeStruct((N, D), jnp.float32),
)


_PERM = [
    32 * j + (2 * k if k < 16 else 2 * (k - 16) + 1)
    for j in range(HIDDEN // 32) for k in range(32)
]


def kernel(x, edge_index, t, W1, b1, Wt, bt, W2, b2):
    row = edge_index[0]
    col = edge_index[1]

    y1, y2 = _proj(
        x, W1, jnp.asarray(t, jnp.float32).reshape(1, 1), Wt,
        b1.reshape(1, HIDDEN), bt.reshape(1, HIDDEN)
    )
    zacc = jnp.zeros((NPAD, AUG), jnp.float32)
    y1 = lax.bitcast_convert_type(y1.reshape(N, HIDDEN // 2, 2), jnp.int32)
    y2 = lax.bitcast_convert_type(y2.reshape(N, HIDDEN // 2, 2), jnp.int32)
    s1, s2 = _edge_kernel(y1, y2, row, col, zacc)
    return _combine(s1, s2, W2[_PERM, :], b2.reshape(1, 2 * D))


# speedup vs baseline: 12.5563x; 1.0949x over previous
import functools

import jax
import jax.numpy as jnp
from jax import lax
from jax.experimental import pallas as pl
from jax.experimental.pallas import tpu as pltpu
from jax.experimental.pallas import tpu_sc as plsc

N = 10000
D = 128
E = 320000
HIDDEN = 64
TEMB = 128
AUG = 80

NC = 2
NS = 16
NW = NC * NS
EPW = E // NW
CH = 40
NCHUNK = EPW // CH
NPAD = 10240
RPT = NPAD // NS
BN = 1000
BNC = 1000

NIB = 6
NGB = 2
NHB = 2
IPD = NIB - 2
MAIN = 240

_HIGH = lax.Precision.HIGHEST


def _pack_bf16_pair(lo, hi):
    ul = lax.bitcast_convert_type(lo, jnp.uint32)
    uh = lax.bitcast_convert_type(hi, jnp.uint32)
    rl = (ul + 0x7FFF + ((ul >> 16) & 1)) >> 16
    rh = (uh + 0x7FFF + ((uh >> 16) & 1)) & jnp.uint32(0xFFFF0000)
    return lax.bitcast_convert_type(rl | rh, jnp.int32)


def _proj_body(x_ref, w1_ref, t_ref, wt_ref, b1_ref, bt_ref, y1_ref, y2_ref):
    half = TEMB // 2
    k = lax.iota(jnp.int32, half).astype(jnp.float32)
    freqs = jnp.exp(-jnp.log(10000.0) * k / (half - 1)).reshape(1, half)
    args = t_ref[0, 0] * freqs
    temb = jnp.concatenate([jnp.sin(args), jnp.cos(args)], axis=-1)
    cvec = (
        jnp.dot(temb, wt_ref[...], preferred_element_type=jnp.float32,
                precision=_HIGH)
        + b1_ref[...]
        + bt_ref[...]
    )
    x = x_ref[...]
    y1 = jnp.dot(x, w1_ref[0:D, :], preferred_element_type=jnp.float32,
                 precision=_HIGH) + cvec
    y2 = jnp.dot(x, w1_ref[D:2 * D, :], preferred_element_type=jnp.float32,
                 precision=_HIGH)
    h2 = HIDDEN // 2
    y1_ref[...] = _pack_bf16_pair(y1[:, 0:h2], y1[:, h2:HIDDEN])
    y2_ref[...] = _pack_bf16_pair(y2[:, 0:h2], y2[:, h2:HIDDEN])


_proj = pl.pallas_call(
    _proj_body,
    grid=(N // BN,),
    in_specs=[
        pl.BlockSpec((BN, D), lambda i: (i, 0)),
        pl.BlockSpec((2 * D, HIDDEN), lambda i: (0, 0)),
        pl.BlockSpec((1, 1), lambda i: (0, 0)),
        pl.BlockSpec((TEMB, HIDDEN), lambda i: (0, 0)),
        pl.BlockSpec((1, HIDDEN), lambda i: (0, 0)),
        pl.BlockSpec((1, HIDDEN), lambda i: (0, 0)),
    ],
    out_specs=[
        pl.BlockSpec((BN, HIDDEN // 2), lambda i: (i, 0)),
        pl.BlockSpec((BN, HIDDEN // 2), lambda i: (i, 0)),
    ],
    out_shape=[
        jax.ShapeDtypeStruct((N, HIDDEN // 2), jnp.int32),
        jax.ShapeDtypeStruct((N, HIDDEN // 2), jnp.int32),
    ],
)


_sc_mesh = plsc.VectorSubcoreMesh(
    core_axis_name="c", subcore_axis_name="s", num_cores=NC, num_subcores=NS
)


@functools.partial(
    pl.kernel,
    out_type=(
        jax.ShapeDtypeStruct((NC, NPAD, AUG), jnp.float32),
        jax.ShapeDtypeStruct((NC, NPAD, AUG), jnp.float32),
    ),
    mesh=_sc_mesh,
    compiler_params=pltpu.CompilerParams(use_tc_tiling_on_sc=False),
    scratch_types=(
        pltpu.VMEM_SHARED((NPAD, AUG), jnp.float32),
        pltpu.VMEM_SHARED((NPAD, AUG), jnp.float32),
        pltpu.VMEM((NIB, CH), jnp.int32),
        pltpu.VMEM((NIB, CH), jnp.int32),
        pltpu.VMEM((NGB, CH, HIDDEN // 2), jnp.int32),
        pltpu.VMEM((NGB, CH, HIDDEN // 2), jnp.int32),
        pltpu.VMEM((NHB, CH, AUG), jnp.float32),
        pltpu.SemaphoreType.DMA((NIB,)),
        pltpu.SemaphoreType.DMA((NGB,)),
        pltpu.SemaphoreType.DMA((NHB,)),
    ),
)
def _edge_kernel(y1_hbm, y2_hbm, row_hbm, col_hbm, zacc_hbm,
                 s1_out, s2_out,
                 acc1, acc2,
                 idx_r, idx_c, abuf, bbuf, hbuf,
                 isem, gsem, ssem):
    cid = lax.axis_index("c")
    sid = lax.axis_index("s")
    wid = sid * NC + cid
    ebase = wid * EPW

    onehot = jnp.where(lax.iota(jnp.int32, 16) == 0,
                       jnp.float32(1.0), jnp.float32(0.0))

    @plsc.parallel_loop(0, NHB * CH, step=1, unroll=8)
    def _init_tail(e):
        hbuf[e // CH, e % CH, pl.ds(HIDDEN, 16)] = onehot

    base = sid * RPT
    pltpu.sync_copy(zacc_hbm.at[pl.ds(base, RPT)], acc1.at[pl.ds(base, RPT)])
    pltpu.sync_copy(zacc_hbm.at[pl.ds(base, RPT)], acc2.at[pl.ds(base, RPT)])
    plsc.subcore_barrier()

    def start_idx(m, q):
        off = ebase + m * CH
        pltpu.async_copy(row_hbm.at[pl.ds(off, CH)], idx_r.at[q], isem.at[q])
        pltpu.async_copy(col_hbm.at[pl.ds(off, CH)], idx_c.at[q], isem.at[q])

    def wait_idx(q):
        pltpu.make_async_copy(
            row_hbm.at[pl.ds(0, CH)], idx_r.at[q], isem.at[q]).wait()
        pltpu.make_async_copy(
            col_hbm.at[pl.ds(0, CH)], idx_c.at[q], isem.at[q]).wait()

    def start_gather(q, g):
        pltpu.async_copy(y1_hbm.at[idx_r.at[q]], abuf.at[g], gsem.at[g])
        pltpu.async_copy(y2_hbm.at[idx_c.at[q]], bbuf.at[g], gsem.at[g])

    def wait_gather(q, g):
        pltpu.make_async_copy(
            y1_hbm.at[idx_r.at[q]], abuf.at[g], gsem.at[g]).wait()
        pltpu.make_async_copy(
            y2_hbm.at[idx_c.at[q]], bbuf.at[g], gsem.at[g]).wait()

    def silu(q, g, h):
        a = abuf.at[g]
        b = bbuf.at[g]
        hb = hbuf.at[h]

        mask = jnp.int32(-65536)

        @plsc.parallel_loop(0, CH, step=1, unroll=8)
        def _silu_row(e):
            for j in range(HIDDEN // 32):
                wa = a[e, pl.ds(j * 16, 16)]
                wb = b[e, pl.ds(j * 16, 16)]
                al = lax.bitcast_convert_type(wa << 16, jnp.float32)
                au = lax.bitcast_convert_type(wa & mask, jnp.float32)
                bl = lax.bitcast_convert_type(wb << 16, jnp.float32)
                bu = lax.bitcast_convert_type(wb & mask, jnp.float32)
                zl = al + bl
                zu = au + bu
                hb[e, pl.ds(j * 16, 16)] = zl / (1.0 + jnp.exp(-zl))
                hb[e, pl.ds(32 + j * 16, 16)] = zu / (1.0 + jnp.exp(-zu))

        pltpu.async_copy(hb, acc1.at[idx_r.at[q]], ssem.at[h], add=True)
        pltpu.async_copy(hb, acc2.at[idx_c.at[q]], ssem.at[h], add=True)

    def wait_scatter(q, h):
        pltpu.make_async_copy(
            hbuf.at[h], acc1.at[idx_r.at[q]], ssem.at[h]).wait()
        pltpu.make_async_copy(
            hbuf.at[h], acc2.at[idx_c.at[q]], ssem.at[h]).wait()

    for m in range(IPD):
        start_idx(m, m)
    wait_idx(0)
    start_gather(0, 0)

    for m in (0, 1):
        wait_idx(m + 1)
        start_gather(m + 1, (m + 1) % NGB)
        wait_gather(m, m % NGB)
        silu(m, m % NGB, m % NHB)
        start_idx(m + IPD, (m + IPD) % NIB)

    @pl.loop(2, 2 + MAIN, step=NIB)
    def _six(k):
        for d in range(NIB):
            m = k + d
            q = (2 + d) % NIB
            g = d % NGB
            h = d % NHB
            wait_idx((q + 1) % NIB)
            start_gather((q + 1) % NIB, (g + 1) % NGB)
            wait_scatter((q - 2) % NIB, h)
            wait_gather(q, g)
            silu(q, g, h)
            start_idx(m + IPD, (q - 2) % NIB)

    for m in range(2 + MAIN, NCHUNK):
        if m + 1 < NCHUNK:
            wait_idx((m + 1) % NIB)
            start_gather((m + 1) % NIB, (m + 1) % NGB)
        wait_scatter((m - 2) % NIB, m % NHB)
        wait_gather(m % NIB, m % NGB)
        silu(m % NIB, m % NGB, m % NHB)
        if m + IPD < NCHUNK:
            start_idx(m + IPD, (m - 2) % NIB)

    wait_scatter((NCHUNK - 2) % NIB, (NCHUNK - 2) % NHB)
    wait_scatter((NCHUNK - 1) % NIB, (NCHUNK - 1) % NHB)

    plsc.subcore_barrier()
    pltpu.sync_copy(acc1.at[pl.ds(base, RPT)], s1_out.at[cid, pl.ds(base, RPT)])
    pltpu.sync_copy(acc2.at[pl.ds(base, RPT)], s2_out.at[cid, pl.ds(base, RPT)])


def _combine_body(s1_ref, s2_ref, w2_ref, b2_ref, out_ref):
    s1 = s1_ref[0] + s1_ref[1]
    s2 = s2_ref[0] + s2_ref[1]
    w2 = w2_ref[...]
    out_ref[...] = (
        jnp.dot(s1[:, 0:HIDDEN], w2[:, 0:D],
                preferred_element_type=jnp.float32, precision=_HIGH)
        + jnp.dot(s2[:, 0:HIDDEN], w2[:, D:2 * D],
                  preferred_element_type=jnp.float32, precision=_HIGH)
        + s1[:, HIDDEN:HIDDEN + 1] * b2_ref[0:1, 0:D]
        + s2[:, HIDDEN:HIDDEN + 1] * b2_ref[0:1, D:2 * D]
    )


_combine = pl.pallas_call(
    _combine_body,
    grid=(N // BNC,),
    in_specs=[
        pl.BlockSpec((NC, BNC, AUG), lambda i: (0, i, 0)),
        pl.BlockSpec((NC, BNC, AUG), lambda i: (0, i, 0)),
        pl.BlockSpec((HIDDEN, 2 * D), lambda i: (0, 0)),
        pl.BlockSpec((1, 2 * D), lambda i: (0, 0)),
    ],
    out_specs=pl.BlockSpec((BNC, D), lambda i: (i, 0)),
    out_shape=jax.ShapeDtypeStruct((N, D), jnp.float32),
)


def kernel(x, edge_index, t, W1, b1, Wt, bt, W2, b2):
    row = edge_index[0]
    col = edge_index[1]

    y1, y2 = _proj(
        x, W1, jnp.asarray(t, jnp.float32).reshape(1, 1), Wt,
        b1.reshape(1, HIDDEN), bt.reshape(1, HIDDEN)
    )
    zacc = jnp.zeros((NPAD, AUG), jnp.float32)
    s1, s2 = _edge_kernel(y1, y2, row, col, zacc)
    return _combine(s1, s2, W2, b2.reshape(1, 2 * D))


# speedup vs baseline: 15.4970x; 1.2342x over previous
import functools

import jax
import jax.numpy as jnp
from jax import lax
from jax.experimental import pallas as pl
from jax.experimental.pallas import tpu as pltpu
from jax.experimental.pallas import tpu_sc as plsc

N = 10000
D = 128
E = 320000
HIDDEN = 64
TEMB = 128
AUG = 80

NC = 2
NS = 16
NW = NC * NS
EPW = E // NW
CH = 80
NCHUNK = EPW // CH
NPAD = 10240
RPT = NPAD // NS
BN = 1000
BNC = 1000

NIB = 6
NGB = 2
NHB = 2
IPD = NIB - 2
MAIN = 114

_HIGH = lax.Precision.HIGHEST


def _pack_bf16_pair(lo, hi):
    ul = lax.bitcast_convert_type(lo, jnp.uint32)
    uh = lax.bitcast_convert_type(hi, jnp.uint32)
    rl = (ul + 0x7FFF + ((ul >> 16) & 1)) >> 16
    rh = (uh + 0x7FFF + ((uh >> 16) & 1)) & jnp.uint32(0xFFFF0000)
    return lax.bitcast_convert_type(rl | rh, jnp.int32)


def _proj_body(x_ref, w1_ref, t_ref, wt_ref, b1_ref, bt_ref, y1_ref, y2_ref):
    half = TEMB // 2
    k = lax.iota(jnp.int32, half).astype(jnp.float32)
    freqs = jnp.exp(-jnp.log(10000.0) * k / (half - 1)).reshape(1, half)
    args = t_ref[0, 0] * freqs
    temb = jnp.concatenate([jnp.sin(args), jnp.cos(args)], axis=-1)
    cvec = (
        jnp.dot(temb, wt_ref[...], preferred_element_type=jnp.float32,
                precision=_HIGH)
        + b1_ref[...]
        + bt_ref[...]
    )
    x = x_ref[...]
    y1 = jnp.dot(x, w1_ref[0:D, :], preferred_element_type=jnp.float32,
                 precision=_HIGH) + cvec
    y2 = jnp.dot(x, w1_ref[D:2 * D, :], preferred_element_type=jnp.float32,
                 precision=_HIGH)
    h2 = HIDDEN // 2
    y1_ref[...] = _pack_bf16_pair(y1[:, 0:h2], y1[:, h2:HIDDEN])
    y2_ref[...] = _pack_bf16_pair(y2[:, 0:h2], y2[:, h2:HIDDEN])


_proj = pl.pallas_call(
    _proj_body,
    grid=(N // BN,),
    in_specs=[
        pl.BlockSpec((BN, D), lambda i: (i, 0)),
        pl.BlockSpec((2 * D, HIDDEN), lambda i: (0, 0)),
        pl.BlockSpec((1, 1), lambda i: (0, 0)),
        pl.BlockSpec((TEMB, HIDDEN), lambda i: (0, 0)),
        pl.BlockSpec((1, HIDDEN), lambda i: (0, 0)),
        pl.BlockSpec((1, HIDDEN), lambda i: (0, 0)),
    ],
    out_specs=[
        pl.BlockSpec((BN, HIDDEN // 2), lambda i: (i, 0)),
        pl.BlockSpec((BN, HIDDEN // 2), lambda i: (i, 0)),
    ],
    out_shape=[
        jax.ShapeDtypeStruct((N, HIDDEN // 2), jnp.int32),
        jax.ShapeDtypeStruct((N, HIDDEN // 2), jnp.int32),
    ],
)


_sc_mesh = plsc.VectorSubcoreMesh(
    core_axis_name="c", subcore_axis_name="s", num_cores=NC, num_subcores=NS
)


@functools.partial(
    pl.kernel,
    out_type=(
        jax.ShapeDtypeStruct((NC, NPAD, AUG), jnp.float32),
        jax.ShapeDtypeStruct((NC, NPAD, AUG), jnp.float32),
    ),
    mesh=_sc_mesh,
    compiler_params=pltpu.CompilerParams(use_tc_tiling_on_sc=False),
    scratch_types=(
        pltpu.VMEM_SHARED((NPAD, AUG), jnp.float32),
        pltpu.VMEM_SHARED((NPAD, AUG), jnp.float32),
        pltpu.VMEM((NIB, CH), jnp.int32),
        pltpu.VMEM((NIB, CH), jnp.int32),
        pltpu.VMEM((NGB, CH, HIDDEN // 2), jnp.int32),
        pltpu.VMEM((NGB, CH, HIDDEN // 2), jnp.int32),
        pltpu.VMEM((NHB, CH, AUG), jnp.float32),
        pltpu.SemaphoreType.DMA((NIB,)),
        pltpu.SemaphoreType.DMA((NGB,)),
        pltpu.SemaphoreType.DMA((NHB,)),
    ),
)
def _edge_kernel(y1_hbm, y2_hbm, row_hbm, col_hbm, zacc_hbm,
                 s1_out, s2_out,
                 acc1, acc2,
                 idx_r, idx_c, abuf, bbuf, hbuf,
                 isem, gsem, ssem):
    cid = lax.axis_index("c")
    sid = lax.axis_index("s")
    wid = sid * NC + cid
    ebase = wid * EPW

    onehot = jnp.where(lax.iota(jnp.int32, 16) == 0,
                       jnp.float32(1.0), jnp.float32(0.0))

    @plsc.parallel_loop(0, NHB * CH, step=1, unroll=8)
    def _init_tail(e):
        hbuf[e // CH, e % CH, pl.ds(HIDDEN, 16)] = onehot

    base = sid * RPT
    pltpu.sync_copy(zacc_hbm.at[pl.ds(base, RPT)], acc1.at[pl.ds(base, RPT)])
    pltpu.sync_copy(zacc_hbm.at[pl.ds(base, RPT)], acc2.at[pl.ds(base, RPT)])
    plsc.subcore_barrier()

    def start_idx(m, q):
        off = ebase + m * CH
        pltpu.async_copy(row_hbm.at[pl.ds(off, CH)], idx_r.at[q], isem.at[q])
        pltpu.async_copy(col_hbm.at[pl.ds(off, CH)], idx_c.at[q], isem.at[q])

    def wait_idx(q):
        pltpu.make_async_copy(
            row_hbm.at[pl.ds(0, CH)], idx_r.at[q], isem.at[q]).wait()
        pltpu.make_async_copy(
            col_hbm.at[pl.ds(0, CH)], idx_c.at[q], isem.at[q]).wait()

    def start_gather(q, g):
        pltpu.async_copy(y1_hbm.at[idx_r.at[q]], abuf.at[g], gsem.at[g])
        pltpu.async_copy(y2_hbm.at[idx_c.at[q]], bbuf.at[g], gsem.at[g])

    def wait_gather(q, g):
        pltpu.make_async_copy(
            y1_hbm.at[idx_r.at[q]], abuf.at[g], gsem.at[g]).wait()
        pltpu.make_async_copy(
            y2_hbm.at[idx_c.at[q]], bbuf.at[g], gsem.at[g]).wait()

    def silu(q, g, h):
        a = abuf.at[g]
        b = bbuf.at[g]
        hb = hbuf.at[h]

        mask = jnp.int32(-65536)

        @plsc.parallel_loop(0, CH, step=1, unroll=8)
        def _silu_row(e):
            for j in range(HIDDEN // 32):
                wa = a[e, pl.ds(j * 16, 16)]
                wb = b[e, pl.ds(j * 16, 16)]
                al = lax.bitcast_convert_type(wa << 16, jnp.float32)
                au = lax.bitcast_convert_type(wa & mask, jnp.float32)
                bl = lax.bitcast_convert_type(wb << 16, jnp.float32)
                bu = lax.bitcast_convert_type(wb & mask, jnp.float32)
                zl = al + bl
                zu = au + bu
                hb[e, pl.ds(j * 16, 16)] = zl / (1.0 + jnp.exp(-zl))
                hb[e, pl.ds(32 + j * 16, 16)] = zu / (1.0 + jnp.exp(-zu))

        pltpu.async_copy(hb, acc1.at[idx_r.at[q]], ssem.at[h], add=True)
        pltpu.async_copy(hb, acc2.at[idx_c.at[q]], ssem.at[h], add=True)

    def wait_scatter(q, h):
        pltpu.make_async_copy(
            hbuf.at[h], acc1.at[idx_r.at[q]], ssem.at[h]).wait()
        pltpu.make_async_copy(
            hbuf.at[h], acc2.at[idx_c.at[q]], ssem.at[h]).wait()

    for m in range(IPD):
        start_idx(m, m)
    wait_idx(0)
    start_gather(0, 0)

    for m in (0, 1):
        wait_idx(m + 1)
        start_gather(m + 1, (m + 1) % NGB)
        wait_gather(m, m % NGB)
        silu(m, m % NGB, m % NHB)
        start_idx(m + IPD, (m + IPD) % NIB)

    @pl.loop(2, 2 + MAIN, step=NIB)
    def _six(k):
        for d in range(NIB):
            m = k + d
            q = (2 + d) % NIB
            g = d % NGB
            h = d % NHB
            wait_idx((q + 1) % NIB)
            start_gather((q + 1) % NIB, (g + 1) % NGB)
            wait_scatter((q - 2) % NIB, h)
            wait_gather(q, g)
            silu(q, g, h)
            start_idx(m + IPD, (q - 2) % NIB)

    for m in range(2 + MAIN, NCHUNK):
        if m + 1 < NCHUNK:
            wait_idx((m + 1) % NIB)
            start_gather((m + 1) % NIB, (m + 1) % NGB)
        wait_scatter((m - 2) % NIB, m % NHB)
        wait_gather(m % NIB, m % NGB)
        silu(m % NIB, m % NGB, m % NHB)
        if m + IPD < NCHUNK:
            start_idx(m + IPD, (m - 2) % NIB)

    wait_scatter((NCHUNK - 2) % NIB, (NCHUNK - 2) % NHB)
    wait_scatter((NCHUNK - 1) % NIB, (NCHUNK - 1) % NHB)

    plsc.subcore_barrier()
    pltpu.sync_copy(acc1.at[pl.ds(base, RPT)], s1_out.at[cid, pl.ds(base, RPT)])
    pltpu.sync_copy(acc2.at[pl.ds(base, RPT)], s2_out.at[cid, pl.ds(base, RPT)])


def _combine_body(s1_ref, s2_ref, w2_ref, b2_ref, out_ref):
    s1 = s1_ref[0] + s1_ref[1]
    s2 = s2_ref[0] + s2_ref[1]
    w2 = w2_ref[...]
    out_ref[...] = (
        jnp.dot(s1[:, 0:HIDDEN], w2[:, 0:D],
                preferred_element_type=jnp.float32, precision=_HIGH)
        + jnp.dot(s2[:, 0:HIDDEN], w2[:, D:2 * D],
                  preferred_element_type=jnp.float32, precision=_HIGH)
        + s1[:, HIDDEN:HIDDEN + 1] * b2_ref[0:1, 0:D]
        + s2[:, HIDDEN:HIDDEN + 1] * b2_ref[0:1, D:2 * D]
    )


_combine = pl.pallas_call(
    _combine_body,
    grid=(N // BNC,),
    in_specs=[
        pl.BlockSpec((NC, BNC, AUG), lambda i: (0, i, 0)),
        pl.BlockSpec((NC, BNC, AUG), lambda i: (0, i, 0)),
        pl.BlockSpec((HIDDEN, 2 * D), lambda i: (0, 0)),
        pl.BlockSpec((1, 2 * D), lambda i: (0, 0)),
    ],
    out_specs=pl.BlockSpec((BNC, D), lambda i: (i, 0)),
    out_shape=jax.ShapeDtypeStruct((N, D), jnp.float32),
)


def kernel(x, edge_index, t, W1, b1, Wt, bt, W2, b2):
    row = edge_index[0]
    col = edge_index[1]

    y1, y2 = _proj(
        x, W1, jnp.asarray(t, jnp.float32).reshape(1, 1), Wt,
        b1.reshape(1, HIDDEN), bt.reshape(1, HIDDEN)
    )
    zacc = jnp.zeros((NPAD, AUG), jnp.float32)
    s1, s2 = _edge_kernel(y1, y2, row, col, zacc)
    return _combine(s1, s2, W2, b2.reshape(1, 2 * D))
